# baseline (speedup 1.0000x reference)
import jax
import jax.numpy as jnp
from jax import lax
from jax.experimental import pallas as pl
from jax.experimental.pallas import tpu as pltpu
from jax.experimental.pallas import tpu_sc as plsc

B, N, C, FF, E, K = 16, 8, 256, 1024, 8, 2
L = B * N

_SC_CORES = 2
_TOK = 16
_T = L // _TOK


def _logits_kernel(x_ref, rw_ref, out_ref):
    lt = jax.lax.dot_general(
        rw_ref[:], x_ref[:], dimension_numbers=(((1,), (1,)), ((), ())),
        preferred_element_type=jnp.float32)
    for t in range(_T):
        out_ref[t] = lt[:, t * _TOK:(t + 1) * _TOK]


def _router_sc(logits_hbm, gates_hbm, lg_v, gt_v):
    cid = lax.axis_index("c")
    sid = lax.axis_index("s")
    wid = sid * _SC_CORES + cid

    @pl.when(wid < _T)
    def _():
        pltpu.sync_copy(logits_hbm.at[wid], lg_v)
        ls = [lg_v[e] for e in range(E)]
        m = ls[0]
        for e in range(1, E):
            m = jnp.maximum(m, ls[e])
        exs = [jnp.exp(l - m) for l in ls]
        s = exs[0]
        for e in range(1, E):
            s = s + exs[e]
        ps = [ex / s for ex in exs]
        p1 = ps[0]
        for e in range(1, E):
            p1 = jnp.maximum(p1, ps[e])
        i1 = jnp.full((_TOK,), float(E), jnp.float32)
        for e in range(E):
            i1 = jnp.minimum(i1, jnp.where(ps[e] == p1, float(e), float(E)))
        pm = [jnp.where(i1 == float(e), -1.0, ps[e]) for e in range(E)]
        p2 = pm[0]
        for e in range(1, E):
            p2 = jnp.maximum(p2, pm[e])
        i2 = jnp.full((_TOK,), float(E), jnp.float32)
        for e in range(E):
            i2 = jnp.minimum(i2, jnp.where(pm[e] == p2, float(e), float(E)))
        denom = p1 + p2 + 1e-9
        for e in range(E):
            sel = jnp.where(i1 == float(e), ps[e], 0.0) + \
                  jnp.where(i2 == float(e), ps[e], 0.0)
            gt_v[e] = sel / denom
        pltpu.sync_copy(gt_v, gates_hbm.at[wid])


_router_call = pl.kernel(
    _router_sc,
    out_type=jax.ShapeDtypeStruct((_T, E, _TOK), jnp.float32),
    mesh=plsc.VectorSubcoreMesh(core_axis_name="c", subcore_axis_name="s"),
    scratch_types=[
        pltpu.VMEM((E, _TOK), jnp.float32),
        pltpu.VMEM((E, _TOK), jnp.float32),
    ],
)

_CHUNKS = [(0, 2), (2, 4), (4, 6), (6, 8)]


def _experts_kernel(x_ref, b1_ref, b2_ref, w1_hbm, w2_hbm, eo_ref,
                    w1_buf, w2_buf, sem1, sem2):
    def copies(ci):
        lo, hi = _CHUNKS[ci]
        sl = pl.ds(lo, hi - lo)
        return (pltpu.make_async_copy(w1_hbm.at[sl], w1_buf.at[sl], sem1.at[ci]),
                pltpu.make_async_copy(w2_hbm.at[sl], w2_buf.at[sl], sem2.at[ci]))

    for ci in range(len(_CHUNKS)):
        for c in copies(ci):
            c.start()

    xf = x_ref[:]
    for ci, (lo, hi) in enumerate(_CHUNKS):
        for c in copies(ci):
            c.wait()
        for e in range(lo, hi):
            h = jax.lax.dot_general(
                xf, w1_buf[e], dimension_numbers=(((1,), (1,)), ((), ())),
                preferred_element_type=jnp.float32) + b1_ref[e][None, :]
            h = jnp.maximum(h, 0.0)
            o = jax.lax.dot_general(
                h, w2_buf[e], dimension_numbers=(((1,), (1,)), ((), ())),
                preferred_element_type=jnp.float32) + b2_ref[e][None, :]
            eo_ref[e] = o


def _combine_kernel(eo_ref, g_ref, out_ref):
    g = g_ref[:]
    acc = g[:, 0:1] * eo_ref[0]
    for e in range(1, E):
        acc = acc + g[:, e:e + 1] * eo_ref[e]
    out_ref[:] = acc


def kernel(x, router_w, w1_all, b1_all, w2_all, b2_all):
    xf = x.reshape(L, C)
    logits_blk = pl.pallas_call(
        _logits_kernel,
        out_shape=jax.ShapeDtypeStruct((_T, E, _TOK), jnp.float32),
    )(xf, router_w)

    gates_blk = _router_call(logits_blk)
    gates = gates_blk.swapaxes(1, 2).reshape(L, E)

    eo = pl.pallas_call(
        _experts_kernel,
        in_specs=[
            pl.BlockSpec(memory_space=pltpu.MemorySpace.VMEM),
            pl.BlockSpec(memory_space=pltpu.MemorySpace.VMEM),
            pl.BlockSpec(memory_space=pltpu.MemorySpace.VMEM),
            pl.BlockSpec(memory_space=pl.ANY),
            pl.BlockSpec(memory_space=pl.ANY),
        ],
        out_specs=pl.BlockSpec(memory_space=pltpu.MemorySpace.VMEM),
        out_shape=jax.ShapeDtypeStruct((E, L, C), jnp.float32),
        scratch_shapes=[
            pltpu.VMEM((E, FF, C), jnp.float32),
            pltpu.VMEM((E, C, FF), jnp.float32),
            pltpu.SemaphoreType.DMA((len(_CHUNKS),)),
            pltpu.SemaphoreType.DMA((len(_CHUNKS),)),
        ],
    )(xf, b1_all, b2_all, w1_all, w2_all)

    out = pl.pallas_call(
        _combine_kernel,
        out_shape=jax.ShapeDtypeStruct((L, C), jnp.float32),
    )(eo, gates)
    return out.reshape(B, N, C)


# speedup vs baseline: 3.3644x; 3.3644x over previous
import jax
import jax.numpy as jnp
from jax.experimental import pallas as pl
from jax.experimental.pallas import tpu as pltpu

B, N, C, FF, E, K = 16, 8, 256, 1024, 8, 2
L = B * N


_CHUNKS = [(0, 3), (3, 5), (5, 7), (7, 8)]


def _moe_kernel(x_ref, rw_ref, b1_ref, b2_ref, w1_hbm, w2_hbm, out_ref,
                w1_buf, w2_buf, sem1, sem2):
    def copies(ci):
        lo, hi = _CHUNKS[ci]
        sl = pl.ds(lo, hi - lo)
        return (pltpu.make_async_copy(w1_hbm.at[sl], w1_buf.at[sl], sem1.at[ci]),
                pltpu.make_async_copy(w2_hbm.at[sl], w2_buf.at[sl], sem2.at[ci]))

    for ci in range(len(_CHUNKS)):
        for c in copies(ci):
            c.start()

    def wait(ci):
        for c in copies(ci):
            c.wait()

    xf = x_ref[:]
    logits = jax.lax.dot_general(
        xf, rw_ref[:], dimension_numbers=(((1,), (1,)), ((), ())),
        preferred_element_type=jnp.float32)
    m = jnp.max(logits, axis=1, keepdims=True)
    ex = jnp.exp(logits - m)
    probs = ex / jnp.sum(ex, axis=1, keepdims=True)
    col = jax.lax.broadcasted_iota(jnp.int32, (L, E), 1)
    p1 = jnp.max(probs, axis=1, keepdims=True)
    i1 = jnp.min(jnp.where(probs == p1, col, E), axis=1, keepdims=True)
    mask1 = col == i1
    pm = jnp.where(mask1, -1.0, probs)
    p2 = jnp.max(pm, axis=1, keepdims=True)
    i2 = jnp.min(jnp.where(pm == p2, col, E), axis=1, keepdims=True)
    mask2 = col == i2
    denom = p1 + p2 + 1e-9
    gates = (jnp.where(mask1, probs, 0.0) + jnp.where(mask2, probs, 0.0)) / denom

    acc = jnp.zeros((L, C), dtype=jnp.float32)
    for ci, (lo, hi) in enumerate(_CHUNKS):
      wait(ci)
      for e in range(lo, hi):
        h = jax.lax.dot_general(
            xf, w1_buf[e], dimension_numbers=(((1,), (1,)), ((), ())),
            preferred_element_type=jnp.float32) + b1_ref[e][None, :]
        h = jnp.maximum(h, 0.0)
        o = jax.lax.dot_general(
            h, w2_buf[e], dimension_numbers=(((1,), (1,)), ((), ())),
            preferred_element_type=jnp.float32) + b2_ref[e][None, :]
        acc = acc + gates[:, e:e + 1] * o
    out_ref[:] = acc


def kernel(x, router_w, w1_all, b1_all, w2_all, b2_all):
    xf = x.reshape(L, C)
    out = pl.pallas_call(
        _moe_kernel,
        in_specs=[
            pl.BlockSpec(memory_space=pltpu.MemorySpace.VMEM),
            pl.BlockSpec(memory_space=pltpu.MemorySpace.VMEM),
            pl.BlockSpec(memory_space=pltpu.MemorySpace.VMEM),
            pl.BlockSpec(memory_space=pltpu.MemorySpace.VMEM),
            pl.BlockSpec(memory_space=pl.ANY),
            pl.BlockSpec(memory_space=pl.ANY),
        ],
        out_specs=pl.BlockSpec(memory_space=pltpu.MemorySpace.VMEM),
        out_shape=jax.ShapeDtypeStruct((L, C), jnp.float32),
        scratch_shapes=[
            pltpu.VMEM((E, FF, C), jnp.float32),
            pltpu.VMEM((E, C, FF), jnp.float32),
            pltpu.SemaphoreType.DMA((len(_CHUNKS),)),
            pltpu.SemaphoreType.DMA((len(_CHUNKS),)),
        ],
    )(xf, router_w, b1_all, b2_all, w1_all, w2_all)
    return out.reshape(B, N, C)


# speedup vs baseline: 3.3712x; 1.0020x over previous
import jax
import jax.numpy as jnp
from jax.experimental import pallas as pl
from jax.experimental.pallas import tpu as pltpu

B, N, C, FF, E, K = 16, 8, 256, 1024, 8, 2
L = B * N


_CHUNKS = [(0, 2), (2, 4), (4, 6), (6, 7), (7, 8)]


def _moe_kernel(x_ref, rw_ref, b1_ref, b2_ref, w1_hbm, w2_hbm, out_ref,
                w1_buf, w2_buf, sem1, sem2):
    def copies(ci):
        lo, hi = _CHUNKS[ci]
        sl = pl.ds(lo, hi - lo)
        return (pltpu.make_async_copy(w1_hbm.at[sl], w1_buf.at[sl], sem1.at[ci]),
                pltpu.make_async_copy(w2_hbm.at[sl], w2_buf.at[sl], sem2.at[ci]))

    for ci in range(len(_CHUNKS)):
        for c in copies(ci):
            c.start()

    def wait(ci):
        for c in copies(ci):
            c.wait()

    xf = x_ref[:]
    logits = jax.lax.dot_general(
        xf, rw_ref[:], dimension_numbers=(((1,), (1,)), ((), ())),
        preferred_element_type=jnp.float32)
    m = jnp.max(logits, axis=1, keepdims=True)
    ex = jnp.exp(logits - m)
    probs = ex / jnp.sum(ex, axis=1, keepdims=True)
    col = jax.lax.broadcasted_iota(jnp.int32, (L, E), 1)
    p1 = jnp.max(probs, axis=1, keepdims=True)
    i1 = jnp.min(jnp.where(probs == p1, col, E), axis=1, keepdims=True)
    mask1 = col == i1
    pm = jnp.where(mask1, -1.0, probs)
    p2 = jnp.max(pm, axis=1, keepdims=True)
    i2 = jnp.min(jnp.where(pm == p2, col, E), axis=1, keepdims=True)
    mask2 = col == i2
    denom = p1 + p2 + 1e-9
    gates = (jnp.where(mask1, probs, 0.0) + jnp.where(mask2, probs, 0.0)) / denom

    acc = jnp.zeros((L, C), dtype=jnp.float32)
    for ci, (lo, hi) in enumerate(_CHUNKS):
      wait(ci)
      for e in range(lo, hi):
        h = jax.lax.dot_general(
            xf, w1_buf[e], dimension_numbers=(((1,), (1,)), ((), ())),
            preferred_element_type=jnp.float32) + b1_ref[e][None, :]
        h = jnp.maximum(h, 0.0)
        o = jax.lax.dot_general(
            h, w2_buf[e], dimension_numbers=(((1,), (1,)), ((), ())),
            preferred_element_type=jnp.float32) + b2_ref[e][None, :]
        acc = acc + gates[:, e:e + 1] * o
    out_ref[:] = acc


def kernel(x, router_w, w1_all, b1_all, w2_all, b2_all):
    xf = x.reshape(L, C)
    out = pl.pallas_call(
        _moe_kernel,
        in_specs=[
            pl.BlockSpec(memory_space=pltpu.MemorySpace.VMEM),
            pl.BlockSpec(memory_space=pltpu.MemorySpace.VMEM),
            pl.BlockSpec(memory_space=pltpu.MemorySpace.VMEM),
            pl.BlockSpec(memory_space=pltpu.MemorySpace.VMEM),
            pl.BlockSpec(memory_space=pl.ANY),
            pl.BlockSpec(memory_space=pl.ANY),
        ],
        out_specs=pl.BlockSpec(memory_space=pltpu.MemorySpace.VMEM),
        out_shape=jax.ShapeDtypeStruct((L, C), jnp.float32),
        scratch_shapes=[
            pltpu.VMEM((E, FF, C), jnp.float32),
            pltpu.VMEM((E, C, FF), jnp.float32),
            pltpu.SemaphoreType.DMA((len(_CHUNKS),)),
            pltpu.SemaphoreType.DMA((len(_CHUNKS),)),
        ],
    )(xf, router_w, b1_all, b2_all, w1_all, w2_all)
    return out.reshape(B, N, C)


# speedup vs baseline: 3.4870x; 1.0344x over previous
import jax
import jax.numpy as jnp
from jax.experimental import pallas as pl
from jax.experimental.pallas import tpu as pltpu

B, N, C, FF, E, K = 16, 8, 256, 1024, 8, 2
L = B * N


_CHUNKS = [(0, 2), (2, 4), (4, 6), (6, 8)]


def _moe_kernel(x_ref, rw_ref, b1_ref, b2_ref, w1_hbm, w2_hbm, out_ref,
                w1_buf, w2_buf, sem1, sem2):
    def copies(ci):
        lo, hi = _CHUNKS[ci]
        sl = pl.ds(lo, hi - lo)
        return (pltpu.make_async_copy(w1_hbm.at[sl], w1_buf.at[sl], sem1.at[ci]),
                pltpu.make_async_copy(w2_hbm.at[sl], w2_buf.at[sl], sem2.at[ci]))

    for ci in range(len(_CHUNKS)):
        for c in copies(ci):
            c.start()

    xf = x_ref[:]
    logits = jax.lax.dot_general(
        xf, rw_ref[:], dimension_numbers=(((1,), (1,)), ((), ())),
        preferred_element_type=jnp.float32)
    m = jnp.max(logits, axis=1, keepdims=True)
    ex = jnp.exp(logits - m)
    probs = ex / jnp.sum(ex, axis=1, keepdims=True)
    col = jax.lax.broadcasted_iota(jnp.int32, (L, E), 1)
    p1 = jnp.max(probs, axis=1, keepdims=True)
    i1 = jnp.min(jnp.where(probs == p1, col, E), axis=1, keepdims=True)
    mask1 = col == i1
    pm = jnp.where(mask1, -1.0, probs)
    p2 = jnp.max(pm, axis=1, keepdims=True)
    i2 = jnp.min(jnp.where(pm == p2, col, E), axis=1, keepdims=True)
    mask2 = col == i2
    denom = p1 + p2 + 1e-9
    gates = (jnp.where(mask1, probs, 0.0) + jnp.where(mask2, probs, 0.0)) / denom

    acc = jnp.zeros((L, C), dtype=jnp.float32)
    for ci, (lo, hi) in enumerate(_CHUNKS):
      c1, c2 = copies(ci)
      c1.wait()
      hs = []
      for e in range(lo, hi):
        h = jax.lax.dot_general(
            xf, w1_buf[e], dimension_numbers=(((1,), (1,)), ((), ())),
            preferred_element_type=jnp.float32) + b1_ref[e][None, :]
        hs.append(jnp.maximum(h, 0.0))
      c2.wait()
      for e in range(lo, hi):
        o = jax.lax.dot_general(
            hs[e - lo], w2_buf[e], dimension_numbers=(((1,), (1,)), ((), ())),
            preferred_element_type=jnp.float32) + b2_ref[e][None, :]
        acc = acc + gates[:, e:e + 1] * o
    out_ref[:] = acc


def kernel(x, router_w, w1_all, b1_all, w2_all, b2_all):
    xf = x.reshape(L, C)
    out = pl.pallas_call(
        _moe_kernel,
        in_specs=[
            pl.BlockSpec(memory_space=pltpu.MemorySpace.VMEM),
            pl.BlockSpec(memory_space=pltpu.MemorySpace.VMEM),
            pl.BlockSpec(memory_space=pltpu.MemorySpace.VMEM),
            pl.BlockSpec(memory_space=pltpu.MemorySpace.VMEM),
            pl.BlockSpec(memory_space=pl.ANY),
            pl.BlockSpec(memory_space=pl.ANY),
        ],
        out_specs=pl.BlockSpec(memory_space=pltpu.MemorySpace.VMEM),
        out_shape=jax.ShapeDtypeStruct((L, C), jnp.float32),
        scratch_shapes=[
            pltpu.VMEM((E, FF, C), jnp.float32),
            pltpu.VMEM((E, C, FF), jnp.float32),
            pltpu.SemaphoreType.DMA((len(_CHUNKS),)),
            pltpu.SemaphoreType.DMA((len(_CHUNKS),)),
        ],
    )(xf, router_w, b1_all, b2_all, w1_all, w2_all)
    return out.reshape(B, N, C)
